# Initial kernel scaffold; baseline (speedup 1.0000x reference)
import jax
import jax.numpy as jnp
from jax.experimental import pallas as pl


def kernel(x, edge_index, edge_weight, params):
    raise NotImplementedError("write your pallas kernel here")


# speedup vs baseline: 106.3579x; 106.3579x over previous
import math

import jax
import jax.numpy as jnp
from jax import lax
from jax.experimental import pallas as pl
from jax.experimental.pallas import tpu as pltpu
from jax.experimental.pallas import tpu_sc as plsc

_N = 512
_F = 64
_H = 64
_E = 8192
_NN = _N * _N
_K = 32
_BI = 128
_NBLK = 2

_NW = 32
_EPT = _E // _NW
_ZPT = _NN // 16


def _mm(a, b):
    dims = (((a.ndim - 1,), (0,)), ((), ()))
    return lax.dot_general(a, b, dims, precision=lax.Precision.HIGHEST,
                           preferred_element_type=jnp.float32)


def _sc_body(ei_hbm, ew_hbm, z_hbm, out_hbm, rowv, colv, wvv, idxb, valb, accw):
    cid = lax.axis_index("c")
    sid = lax.axis_index("s")
    wid = sid * 2 + cid
    eb = wid * _EPT
    zb = sid * _ZPT
    pltpu.sync_copy(z_hbm.at[pl.ds(zb, _ZPT)], accw.at[pl.ds(zb, _ZPT)])
    pltpu.sync_copy(ei_hbm.at[0, pl.ds(eb, _EPT)], rowv)
    pltpu.sync_copy(ei_hbm.at[1, pl.ds(eb, _EPT)], colv)
    pltpu.sync_copy(ew_hbm.at[pl.ds(eb, _EPT)], wvv)
    for i in range(_EPT // 16):
        j, off = divmod(i, 8)
        c = colv[pl.ds(i * 16, 16)]
        r = rowv[pl.ds(i * 16, 16)]
        idxb[j, pl.ds(off * 16, 16)] = c * _N + r
        valb[j, pl.ds(off * 16, 16)] = wvv[pl.ds(i * 16, 16)]
    plsc.subcore_barrier()
    for j in range(2):
        pltpu.sync_copy(valb.at[j], accw.at[idxb.at[j]], add=True)
    plsc.subcore_barrier()
    pltpu.sync_copy(accw.at[pl.ds(zb, _ZPT)], out_hbm.at[cid, pl.ds(zb, _ZPT)])


def _build_w(edge_index, edge_weight, zeros):
    mesh = plsc.VectorSubcoreMesh(core_axis_name="c", subcore_axis_name="s")
    return pl.kernel(
        _sc_body,
        out_type=jax.ShapeDtypeStruct((2, _NN), jnp.float32),
        mesh=mesh,
        scratch_types=[
            pltpu.VMEM((_EPT,), jnp.int32),
            pltpu.VMEM((_EPT,), jnp.int32),
            pltpu.VMEM((_EPT,), jnp.float32),
            pltpu.VMEM((2, 128), jnp.int32),
            pltpu.VMEM((2, 128), jnp.float32),
            pltpu.VMEM_SHARED((_NN,), jnp.float32),
        ],
    )(edge_index, edge_weight, zeros)


def _tconv(h, w1, b1, w2, b2, w3, b3):
    p = _mm(h, w1[...]) + b1[...]
    q = jax.nn.sigmoid(_mm(h, w2[...]) + b2[...])
    r = _mm(h, w3[...]) + b3[...]
    return jnp.maximum(p * q + r, 0.0)


def _gnn_body(*refs):
    it = iter(refs)
    wp = next(it)
    x = next(it)
    inw = next(it)
    inb = next(it)
    blocks = []
    for _ in range(_NBLK):
        blocks.append([next(it) for _ in range(19)])
    emb_ref = next(it)
    embt_ref = next(it)

    wm = wp[0] + wp[1]
    ones_col = jnp.ones((_N, 1), jnp.float32)
    deg = lax.dot_general(wm, ones_col, (((0,), (0,)), ((), ())),
                          precision=lax.Precision.HIGHEST,
                          preferred_element_type=jnp.float32)
    pos = deg > 0.0
    dinv = jnp.where(pos, 1.0 / jnp.sqrt(jnp.where(pos, deg, 1.0)), 0.0)

    h = _mm(x[0], inw[...]) + inb[...]
    for blk in blocks:
        (t1w1, t1b1, t1w2, t1b2, t1w3, t1b3,
         cw0, cw1, cb,
         t2w1, t2b1, t2w2, t2b2, t2w3, t2b3,
         bng, bnb, lng, lnb) = blk
        t0 = _tconv(h, t1w1, t1b1, t1w2, t1b2, t1w3, t1b3)
        u = t0 * dinv
        y = _mm(wm, u)
        sx = -(dinv * y)
        t1 = jnp.maximum(_mm(t0, cw0[...]) + _mm(sx, cw1[...]) + cb[...], 0.0)
        t2 = _tconv(t1, t2w1, t2b1, t2w2, t2b2, t2w3, t2b3)
        t2 = t2 * (bng[...] * (1.0 / math.sqrt(1.0 + 1e-5))) + bnb[...]
        m = jnp.mean(t2, axis=-1, keepdims=True)
        v = jnp.mean((t2 - m) ** 2, axis=-1, keepdims=True)
        t2 = (t2 - m) / jnp.sqrt(v + 1e-5) * lng[...] + lnb[...]
        h = h + t2

    emb_ref[0] = h
    rr = lax.broadcasted_iota(jnp.int32, (_H, _H), 0)
    cc = lax.broadcasted_iota(jnp.int32, (_H, _H), 1)
    eye = (rr == cc).astype(jnp.float32)
    embt_ref[0] = lax.dot_general(eye, h, (((1,), (1,)), ((), ())),
                                  precision=lax.Precision.HIGHEST,
                                  preferred_element_type=jnp.float32)


def _gnn(bsz, wp, xe, flat):
    def full(a):
        nd = a.ndim
        return pl.BlockSpec(a.shape, lambda b, _n=nd: (0,) * _n)

    in_specs = [full(wp),
                pl.BlockSpec((1, _N, _F), lambda b: (b, 0, 0))]
    in_specs += [full(a) for a in flat]
    out_specs = [pl.BlockSpec((1, _N, _H), lambda b: (b, 0, 0)),
                 pl.BlockSpec((1, _H, _N), lambda b: (b, 0, 0))]
    return pl.pallas_call(
        _gnn_body,
        grid=(bsz,),
        in_specs=in_specs,
        out_specs=out_specs,
        out_shape=[jax.ShapeDtypeStruct((bsz, _N, _H), jnp.float32),
                   jax.ShapeDtypeStruct((bsz, _H, _N), jnp.float32)],
    )(wp, xe, *flat)


def _dec_body(emb, embt, wa, wbt, o1b, olng, olnb, o2w, o2b, out_ref):
    e = emb[0]
    et = embt[0]
    a = _mm(e, wa[...]) + o1b[...]
    bt = _mm(wbt[...], et)
    s1 = jnp.zeros((_BI, _N), jnp.float32)
    s2 = jnp.zeros((_BI, _N), jnp.float32)
    sg = jnp.zeros((_BI, _N), jnp.float32)
    gtot = 0.0
    cb = 0.0
    for k in range(_K):
        zk = jnp.maximum(a[:, k:k + 1] + bt[k:k + 1, :], 0.0)
        gk = olng[0, k] * o2w[0, k]
        s1 = s1 + zk
        s2 = s2 + zk * zk
        sg = sg + gk * zk
        gtot = gtot + gk
        cb = cb + olnb[0, k] * o2w[0, k]
    m = s1 * (1.0 / _K)
    v = s2 * (1.0 / _K) - m * m
    inv = 1.0 / jnp.sqrt(v + 1e-5)
    out_ref[0] = jax.nn.sigmoid((sg - gtot * m) * inv + (cb + o2b[0, 0]))


def _decode(bsz, emb, embt, wa, wbt, o1b, olng, olnb, o2w, o2b):
    smem = pltpu.MemorySpace.SMEM
    in_specs = [
        pl.BlockSpec((1, _BI, _H), lambda b, i: (b, i, 0)),
        pl.BlockSpec((1, _H, _N), lambda b, i: (b, 0, 0)),
        pl.BlockSpec((_H, _K), lambda b, i: (0, 0)),
        pl.BlockSpec((_K, _H), lambda b, i: (0, 0)),
        pl.BlockSpec((1, _K), lambda b, i: (0, 0)),
        pl.BlockSpec(memory_space=smem),
        pl.BlockSpec(memory_space=smem),
        pl.BlockSpec(memory_space=smem),
        pl.BlockSpec(memory_space=smem),
    ]
    return pl.pallas_call(
        _dec_body,
        grid=(bsz, _N // _BI),
        in_specs=in_specs,
        out_specs=pl.BlockSpec((1, _BI, _N), lambda b, i: (b, i, 0)),
        out_shape=jax.ShapeDtypeStruct((bsz, _N, _N), jnp.float32),
    )(emb, embt, wa, wbt, o1b, olng, olnb, o2w, o2b)


def _flatten_params(p):
    flat = [p['in_w'], p['in_b'].reshape(1, _H)]
    for blk in p['blocks']:
        flat += [
            blk['t1c1_w'], blk['t1c1_b'].reshape(1, _H),
            blk['t1c2_w'], blk['t1c2_b'].reshape(1, _H),
            blk['t1c3_w'], blk['t1c3_b'].reshape(1, _H),
            blk['cheb_w0'], blk['cheb_w1'], blk['cheb_b'].reshape(1, _H),
            blk['t2c1_w'], blk['t2c1_b'].reshape(1, _H),
            blk['t2c2_w'], blk['t2c2_b'].reshape(1, _H),
            blk['t2c3_w'], blk['t2c3_b'].reshape(1, _H),
            blk['bn_g'].reshape(_N, 1), blk['bn_b'].reshape(_N, 1),
            blk['ln_g'].reshape(1, _H), blk['ln_b'].reshape(1, _H),
        ]
    return flat


def kernel(x, edge_index, edge_weight, params):
    bsz = x.shape[0]
    xe = x[:, -1]
    zeros = jnp.zeros((_NN,), jnp.float32)
    wpart = _build_w(edge_index, edge_weight, zeros)
    wp = wpart.reshape(2, _N, _N)
    emb, embt = _gnn(bsz, wp, xe, _flatten_params(params))
    p = params
    return _decode(
        bsz, emb, embt,
        p['o1_w'][:_H],
        p['o1_w'][_H:].T,
        p['o1_b'].reshape(1, _K),
        p['oln_g'].reshape(1, _K),
        p['oln_b'].reshape(1, _K),
        p['o2_w'].reshape(1, _K),
        p['o2_b'].reshape(1, 1),
    )
